# baseline (speedup 1.0000x reference)
import jax
import jax.numpy as jnp
from jax import lax
from jax.experimental import pallas as pl
from jax.experimental.pallas import tpu as pltpu
from jax.experimental.pallas import tpu_sc as plsc

_B, _S, _H = 4, 2048, 4096
_HH = _H // 2
_NC, _NS = 2, 16
_NW = _NC * _NS
_NPG = _NW // 2
_PW = _S // _NPG
_W = 8
_NPC = _PW // _W
_LANES = 16
_UNROLL = 8


def _add_pos(rows, pos):
    @pl.loop(0, _W)
    def _(r):
        @pl.loop(0, _HH, step=_LANES * _UNROLL)
        def _(c):
            vals = [pos[r, pl.ds(c + _LANES * u, _LANES)] for u in range(_UNROLL)]
            for u in range(_UNROLL):
                plsc.addupdate(rows.at[r, pl.ds(c + _LANES * u, _LANES)], vals[u])


def _body(x2_hbm, tok_hbm, pos_hbm, out_hbm,
          idx_v, pos0, pos1, rows0, rows1,
          gsem0, gsem1, ssem0, ssem1, psem0, psem1):
    wid = lax.axis_index("c") * _NS + lax.axis_index("s")
    h = wid & 1
    pg = wid >> 1
    p0 = pg * _PW
    c0 = h * _HH
    rows = (rows0, rows1)
    gsem = (gsem0, gsem1)
    ssem = (ssem0, ssem1)
    pos = (pos0, pos1)
    psem = (psem0, psem1)

    for b in range(_B):
        pltpu.sync_copy(x2_hbm.at[pl.ds(h * (_B * _S) + b * _S + p0, _PW)],
                        idx_v.at[pl.ds(b * _PW, _PW)])

    def g_desc(pc, b, buf):
        return pltpu.make_async_copy(
            tok_hbm.at[idx_v.at[pl.ds(b * _PW + pc * _W, _W)]],
            rows[buf], gsem[buf])

    def s_desc(pc, b, buf):
        return pltpu.make_async_copy(
            rows[buf],
            out_hbm.at[pl.ds(b * _S + p0 + pc * _W, _W), pl.ds(c0, _HH)],
            ssem[buf])

    def p_desc(pc, pbuf):
        return pltpu.make_async_copy(
            pos_hbm.at[pl.ds(p0 + pc * _W, _W), pl.ds(c0, _HH)],
            pos[pbuf], psem[pbuf])

    def items(pc, pbuf):
        p_desc(pc, pbuf).wait()
        for b in range(_B):
            buf = b % 2
            other = 1 - buf
            if b == 0:
                @pl.when(pc > 0)
                def _():
                    s_desc(pc, 3, other).wait()
            else:
                s_desc(pc, b - 1, other).wait()
            if b < _B - 1:
                g_desc(pc, b + 1, other).start()
            else:
                @pl.when(pc < _NPC - 1)
                def _():
                    g_desc(pc + 1, 0, other).start()
            g_desc(pc, b, buf).wait()
            _add_pos(rows[buf], pos[pbuf])
            s_desc(pc, b, buf).start()

    p_desc(0, 0).start()
    g_desc(0, 0, 0).start()

    @pl.loop(0, _NPC, step=2)
    def _(pc):
        p_desc(pc + 1, 1).start()
        items(pc, 0)
        @pl.when(pc + 2 < _NPC)
        def _():
            p_desc(pc + 2, 0).start()
        items(pc + 1, 1)

    s_desc(_NPC - 1, _B - 1, 1).wait()


_emb_call = pl.kernel(
    _body,
    out_type=jax.ShapeDtypeStruct((_B * _S, _H), jnp.float32),
    mesh=plsc.VectorSubcoreMesh(core_axis_name="c", subcore_axis_name="s"),
    scratch_types=[
        pltpu.VMEM((2 * _B * _PW,), jnp.int32),
        pltpu.VMEM((_W, _HH), jnp.float32),
        pltpu.VMEM((_W, _HH), jnp.float32),
        pltpu.VMEM((_W, _HH), jnp.float32),
        pltpu.VMEM((_W, _HH), jnp.float32),
        pltpu.SemaphoreType.DMA,
        pltpu.SemaphoreType.DMA,
        pltpu.SemaphoreType.DMA,
        pltpu.SemaphoreType.DMA,
        pltpu.SemaphoreType.DMA,
        pltpu.SemaphoreType.DMA,
    ],
)


@jax.jit
def _emb(x_flat, tok_emb, pos_emb):
    tok2 = tok_emb.reshape(2 * tok_emb.shape[0], _HH)
    x2 = jnp.concatenate([2 * x_flat, 2 * x_flat + 1])
    return _emb_call(x2, tok2, pos_emb)


def kernel(x, tok_emb, pos_emb):
    x_flat = x.reshape(-1).astype(jnp.int32)
    out = _emb(x_flat, tok_emb, pos_emb)
    return out.reshape(_B, _S, _H)


# speedup vs baseline: 11.9266x; 11.9266x over previous
import jax
import jax.numpy as jnp
from jax import lax
from jax.experimental import pallas as pl
from jax.experimental.pallas import tpu as pltpu
from jax.experimental.pallas import tpu_sc as plsc

_B, _S, _H = 4, 2048, 4096
_HH = _H // 2
_NC, _NS = 2, 16
_NW = _NC * _NS
_NPG = _NW // 2
_PW = _S // _NPG
_W = 8
_NPC = _PW // _W
_LANES = 16
_UNROLL = 8


def _add_pos(rows, pos):
    @pl.loop(0, _W)
    def _(r):
        @pl.loop(0, _HH, step=_LANES * _UNROLL)
        def _(c):
            vals = [pos[r, pl.ds(c + _LANES * u, _LANES)] for u in range(_UNROLL)]
            for u in range(_UNROLL):
                plsc.addupdate(rows.at[r, pl.ds(c + _LANES * u, _LANES)], vals[u])


def _body(x2_hbm, tok_hbm, pos_hbm, out_hbm,
          idx_v, pos0, pos1, rows0, rows1,
          gsem0, gsem1, ssem0, ssem1, psem0, psem1):
    wid = lax.axis_index("c") * _NS + lax.axis_index("s")
    h = wid & 1
    pg = wid >> 1
    p0 = pg * _PW
    c0 = h * _HH
    rows = (rows0, rows1)
    gsem = (gsem0, gsem1)
    ssem = (ssem0, ssem1)
    pos = (pos0, pos1)
    psem = (psem0, psem1)

    for b in range(_B):
        pltpu.sync_copy(x2_hbm.at[pl.ds(b * _S + p0, _PW)],
                        idx_v.at[pl.ds(b * _PW, _PW)])

    def g_desc(pc, b, buf):
        return pltpu.make_async_copy(
            tok_hbm.at[idx_v.at[pl.ds(b * _PW + pc * _W, _W)], pl.ds(c0, _HH)],
            rows[buf], gsem[buf])

    def s_desc(pc, b, buf):
        return pltpu.make_async_copy(
            rows[buf],
            out_hbm.at[pl.ds(b * _S + p0 + pc * _W, _W), pl.ds(c0, _HH)],
            ssem[buf])

    def p_desc(pc, pbuf):
        return pltpu.make_async_copy(
            pos_hbm.at[pl.ds(p0 + pc * _W, _W), pl.ds(c0, _HH)],
            pos[pbuf], psem[pbuf])

    def items(pc, pbuf):
        p_desc(pc, pbuf).wait()
        for b in range(_B):
            buf = b % 2
            other = 1 - buf
            if b == 0:
                @pl.when(pc > 0)
                def _():
                    s_desc(pc, 3, other).wait()
            else:
                s_desc(pc, b - 1, other).wait()
            if b < _B - 1:
                g_desc(pc, b + 1, other).start()
            else:
                @pl.when(pc < _NPC - 1)
                def _():
                    g_desc(pc + 1, 0, other).start()
            g_desc(pc, b, buf).wait()
            _add_pos(rows[buf], pos[pbuf])
            s_desc(pc, b, buf).start()

    p_desc(0, 0).start()
    g_desc(0, 0, 0).start()

    @pl.loop(0, _NPC, step=2)
    def _(pc):
        p_desc(pc + 1, 1).start()
        items(pc, 0)
        @pl.when(pc + 2 < _NPC)
        def _():
            p_desc(pc + 2, 0).start()
        items(pc + 1, 1)

    s_desc(_NPC - 1, _B - 1, 1).wait()


_emb_call = pl.kernel(
    _body,
    out_type=jax.ShapeDtypeStruct((_B * _S, _H), jnp.float32),
    mesh=plsc.VectorSubcoreMesh(core_axis_name="c", subcore_axis_name="s"),
    scratch_types=[
        pltpu.VMEM((_B * _PW,), jnp.int32),
        pltpu.VMEM((_W, _HH), jnp.float32),
        pltpu.VMEM((_W, _HH), jnp.float32),
        pltpu.VMEM((_W, _HH), jnp.float32),
        pltpu.VMEM((_W, _HH), jnp.float32),
        pltpu.SemaphoreType.DMA,
        pltpu.SemaphoreType.DMA,
        pltpu.SemaphoreType.DMA,
        pltpu.SemaphoreType.DMA,
        pltpu.SemaphoreType.DMA,
        pltpu.SemaphoreType.DMA,
    ],
)


@jax.jit
def _emb(x_flat, tok_emb, pos_emb):
    return _emb_call(x_flat, tok_emb, pos_emb)


def kernel(x, tok_emb, pos_emb):
    x_flat = x.reshape(-1).astype(jnp.int32)
    out = _emb(x_flat, tok_emb, pos_emb)
    return out.reshape(_B, _S, _H)


# speedup vs baseline: 13.7162x; 1.1501x over previous
import jax
import jax.numpy as jnp
from jax import lax
from jax.experimental import pallas as pl
from jax.experimental.pallas import tpu as pltpu
from jax.experimental.pallas import tpu_sc as plsc

_B, _S, _H = 4, 2048, 4096
_HH = _H // 4
_NC, _NS = 2, 16
_NW = _NC * _NS
_NPG = _NW // 4
_PW = _S // _NPG
_W = 8
_NPC = _PW // _W
_LANES = 16
_UNROLL = 8


def _add_pos4(r0, r1, r2, r3, pos):
    @pl.loop(0, _W)
    def _(r):
        @pl.loop(0, _HH, step=_LANES * _UNROLL)
        def _(c):
            vals = [pos[r, pl.ds(c + _LANES * u, _LANES)] for u in range(_UNROLL)]
            for u in range(_UNROLL):
                for rb in (r0, r1, r2, r3):
                    plsc.addupdate(rb.at[r, pl.ds(c + _LANES * u, _LANES)], vals[u])


def _body(x_hbm, tok_hbm, pos_hbm, out_hbm,
          idx_v, pos0, pos1,
          ra0, rb0, rc0, rd0, ra1, rb1, rc1, rd1,
          ga0, gb0, gc0, gd0, ga1, gb1, gc1, gd1,
          sa0, sb0, sc0, sd0, sa1, sb1, sc1, sd1,
          psem0, psem1):
    wid = lax.axis_index("c") * _NS + lax.axis_index("s")
    h = wid & 3
    pg = wid >> 2
    p0 = pg * _PW
    c0 = h * _HH
    rows = ((ra0, rb0, rc0, rd0), (ra1, rb1, rc1, rd1))
    gsem = ((ga0, gb0, gc0, gd0), (ga1, gb1, gc1, gd1))
    ssem = ((sa0, sb0, sc0, sd0), (sa1, sb1, sc1, sd1))
    pos = (pos0, pos1)
    psem = (psem0, psem1)

    for b in range(_B):
        pltpu.sync_copy(x_hbm.at[pl.ds(b * _S + p0, _PW)],
                        idx_v.at[pl.ds(b * _PW, _PW)])

    def g_desc(pc, b, d):
        return pltpu.make_async_copy(
            tok_hbm.at[idx_v.at[pl.ds(b * _PW + pc * _W, _W)], pl.ds(c0, _HH)],
            rows[d][b], gsem[d][b])

    def s_desc(pc, b, d):
        return pltpu.make_async_copy(
            rows[d][b],
            out_hbm.at[pl.ds(b * _S + p0 + pc * _W, _W), pl.ds(c0, _HH)],
            ssem[d][b])

    def p_desc(pc, pd):
        return pltpu.make_async_copy(
            pos_hbm.at[pl.ds(p0 + pc * _W, _W), pl.ds(c0, _HH)],
            pos[pd], psem[pd])

    def chunk(pc, d, guard_drain, guard_gather):
        def drain():
            for b in range(_B):
                s_desc(pc - 1, b, d ^ 1).wait()

        def prefetch():
            for b in range(_B):
                g_desc(pc + 1, b, d ^ 1).start()

        if guard_drain:
            pl.when(pc > 0)(drain)
        else:
            drain()
        if guard_gather:
            pl.when(pc + 1 < _NPC)(prefetch)
        else:
            prefetch()
        for b in range(_B):
            g_desc(pc, b, d).wait()
        p_desc(pc, d).wait()
        _add_pos4(rows[d][0], rows[d][1], rows[d][2], rows[d][3], pos[d])
        for b in range(_B):
            s_desc(pc, b, d).start()

    p_desc(0, 0).start()
    for b in range(_B):
        g_desc(0, b, 0).start()

    @pl.loop(0, _NPC, step=2)
    def _(pc):
        p_desc(pc + 1, 1).start()
        chunk(pc, 0, True, False)

        @pl.when(pc + 2 < _NPC)
        def _():
            p_desc(pc + 2, 0).start()
        chunk(pc + 1, 1, False, True)

    for b in range(_B):
        s_desc(_NPC - 1, b, 1).wait()


_emb_call = pl.kernel(
    _body,
    out_type=jax.ShapeDtypeStruct((_B * _S, _H), jnp.float32),
    mesh=plsc.VectorSubcoreMesh(core_axis_name="c", subcore_axis_name="s"),
    scratch_types=[
        pltpu.VMEM((_B * _PW,), jnp.int32),
        pltpu.VMEM((_W, _HH), jnp.float32),
        pltpu.VMEM((_W, _HH), jnp.float32),
        pltpu.VMEM((_W, _HH), jnp.float32),
        pltpu.VMEM((_W, _HH), jnp.float32),
        pltpu.VMEM((_W, _HH), jnp.float32),
        pltpu.VMEM((_W, _HH), jnp.float32),
        pltpu.VMEM((_W, _HH), jnp.float32),
        pltpu.VMEM((_W, _HH), jnp.float32),
        pltpu.VMEM((_W, _HH), jnp.float32),
        pltpu.VMEM((_W, _HH), jnp.float32),
        pltpu.SemaphoreType.DMA,
        pltpu.SemaphoreType.DMA,
        pltpu.SemaphoreType.DMA,
        pltpu.SemaphoreType.DMA,
        pltpu.SemaphoreType.DMA,
        pltpu.SemaphoreType.DMA,
        pltpu.SemaphoreType.DMA,
        pltpu.SemaphoreType.DMA,
        pltpu.SemaphoreType.DMA,
        pltpu.SemaphoreType.DMA,
        pltpu.SemaphoreType.DMA,
        pltpu.SemaphoreType.DMA,
        pltpu.SemaphoreType.DMA,
        pltpu.SemaphoreType.DMA,
        pltpu.SemaphoreType.DMA,
        pltpu.SemaphoreType.DMA,
        pltpu.SemaphoreType.DMA,
        pltpu.SemaphoreType.DMA,
    ],
)


@jax.jit
def _emb(x_flat, tok_emb, pos_emb):
    return _emb_call(x_flat, tok_emb, pos_emb)


def kernel(x, tok_emb, pos_emb):
    x_flat = x.reshape(-1).astype(jnp.int32)
    out = _emb(x_flat, tok_emb, pos_emb)
    return out.reshape(_B, _S, _H)
